# baseline (speedup 1.0000x reference)
import functools

import jax
import jax.numpy as jnp
from jax import lax
from jax.experimental import pallas as pl
from jax.experimental.pallas import tpu as pltpu
from jax.experimental.pallas import tpu_sc as plsc

EMB_DIM = 64
CHUNK = 128


@functools.cache
def _make_gather(n_rows: int, n_emb: int):
  NC, NS = 2, 16
  NW = NC * NS
  assert n_rows % (NW * CHUNK) == 0
  ch_per_w = n_rows // (NW * CHUNK)

  mesh = plsc.VectorSubcoreMesh(core_axis_name="c", subcore_axis_name="s")

  @functools.partial(
      pl.kernel,
      out_type=jax.ShapeDtypeStruct((n_rows, EMB_DIM), jnp.float32),
      mesh=mesh,
      scratch_types=[
          pltpu.VMEM((ch_per_w, CHUNK), jnp.int32),
          pltpu.VMEM((2, CHUNK, EMB_DIM), jnp.float32),
          pltpu.SemaphoreType.DMA((2,)),
          pltpu.SemaphoreType.DMA((2,)),
      ],
      compiler_params=pltpu.CompilerParams(use_tc_tiling_on_sc=False),
  )
  def grab(idx_hbm, table_hbm, out_hbm, idx_v, rows_v, gsem, osem):
    wid = lax.axis_index("s") * NC + lax.axis_index("c")
    base_chunk = wid * ch_per_w
    pltpu.sync_copy(idx_hbm.at[pl.ds(base_chunk, ch_per_w)], idx_v)

    def gather_start(c, buf):
      pltpu.async_copy(table_hbm.at[idx_v.at[c]], rows_v.at[buf], gsem.at[buf])

    def store_start(c, buf):
      pltpu.async_copy(
          rows_v.at[buf],
          out_hbm.at[pl.ds((base_chunk + c) * CHUNK, CHUNK)],
          osem.at[buf],
      )

    gather_start(0, 0)

    @pl.loop(0, ch_per_w)
    def _(c):
      buf = lax.rem(c, 2)
      nbuf = 1 - buf

      @pl.when(c >= 1)
      def _():
        pltpu.make_async_copy(
            rows_v.at[nbuf],
            out_hbm.at[pl.ds((base_chunk + c - 1) * CHUNK, CHUNK)],
            osem.at[nbuf],
        ).wait()

      @pl.when(c + 1 < ch_per_w)
      def _():
        gather_start(c + 1, nbuf)

      pltpu.make_async_copy(
          table_hbm.at[idx_v.at[c]], rows_v.at[buf], gsem.at[buf]
      ).wait()
      store_start(c, buf)

    last = ch_per_w - 1
    pltpu.make_async_copy(
        rows_v.at[lax.rem(last, 2)],
        out_hbm.at[pl.ds((base_chunk + last) * CHUNK, CHUNK)],
        osem.at[lax.rem(last, 2)],
    ).wait()

  return grab


def kernel(input, embedding_weight):
  b, f = input.shape
  n_rows = b * f
  idx = input.astype(jnp.int32).reshape(n_rows // CHUNK, CHUNK)
  grab = _make_gather(n_rows, embedding_weight.shape[0])
  out = grab(idx, embedding_weight)
  return out.astype(jnp.bfloat16).reshape(b, f, EMB_DIM)
